# baseline (speedup 1.0000x reference)
import functools

import jax
import jax.numpy as jnp
from jax import lax
from jax.experimental import pallas as pl
from jax.experimental.pallas import tpu as pltpu
from jax.experimental.pallas import tpu_sc as plsc

D_MODEL = 1024
BATCH = 4
SEQ = 2048
N_TOKENS = BATCH * SEQ

_info = plsc.get_sparse_core_info()
NC, NS = _info.num_cores, _info.num_subcores
NW = NC * NS
B_PER_W = N_TOKENS // NW
W_PER_ROW = SEQ // B_PER_W
CHUNK = 32
NCHUNK = B_PER_W // CHUNK
NPAIR = NCHUNK // 2

_mesh = plsc.VectorSubcoreMesh(core_axis_name="c", subcore_axis_name="s")


@functools.partial(
    pl.kernel,
    mesh=_mesh,
    out_type=jax.ShapeDtypeStruct((N_TOKENS, D_MODEL), jnp.float32),
    scratch_types=[
        pltpu.VMEM((B_PER_W,), jnp.int32),
        pltpu.VMEM((2, CHUNK, D_MODEL), jnp.float32),
        pltpu.SemaphoreType.DMA,
        pltpu.SemaphoreType.DMA,
        pltpu.SemaphoreType.DMA,
        pltpu.SemaphoreType.DMA,
    ],
)
def _sc_gather(ids_hbm, table_hbm, out_hbm, idx_v, rows_v, gs0, gs1, ss0, ss1):
    wid = lax.axis_index("s") * NC + lax.axis_index("c")
    base = wid * B_PER_W
    row = wid // W_PER_ROW
    col = (wid % W_PER_ROW) * B_PER_W
    pltpu.sync_copy(ids_hbm.at[row, pl.ds(col, B_PER_W)], idx_v)

    gsem = (gs0, gs1)
    ssem = (ss0, ss1)

    def gather(i, b, sem):
        return pltpu.async_copy(
            table_hbm.at[idx_v.at[pl.ds(i * CHUNK, CHUNK)]],
            rows_v.at[b], sem)

    def scatter(i, b, sem):
        return pltpu.async_copy(
            rows_v.at[b], out_hbm.at[pl.ds(base + i * CHUNK, CHUNK)], sem)

    def wait_gather(b):
        pltpu.make_async_copy(
            table_hbm.at[pl.ds(0, CHUNK)], rows_v.at[b], gsem[b]).wait()

    def wait_scatter(b):
        pltpu.make_async_copy(
            rows_v.at[b], out_hbm.at[pl.ds(0, CHUNK)], ssem[b]).wait()

    gather(0, 0, gsem[0])

    def body(j, _):
        i0 = 2 * j
        @pl.when(j >= 1)
        def _():
            wait_scatter(1)
        gather(i0 + 1, 1, gsem[1])
        wait_gather(0)
        scatter(i0, 0, ssem[0])
        @pl.when(j <= NPAIR - 2)
        def _():
            wait_scatter(0)
            gather(i0 + 2, 0, gsem[0])
        wait_gather(1)
        scatter(i0 + 1, 1, ssem[1])
        return _

    lax.fori_loop(0, NPAIR, body, None)
    wait_scatter(0)
    wait_scatter(1)


def kernel(input_ids, embed_table):
    out = _sc_gather(input_ids.astype(jnp.int32), embed_table)
    return out.reshape(BATCH, SEQ, D_MODEL)
